# baseline (speedup 1.0000x reference)
import jax
import jax.numpy as jnp
import numpy as np
from jax.experimental import pallas as pl
from jax.experimental.pallas import tpu as pltpu

_EPS = 1e-5


def _conv_out(s):
    return (s - 5) // 2 + 1


def _fold_layer(w, b, gamma, beta, mean, var, width, compute_dtype,
                c_major=False, width_pad=None, wo_pad=None):
    wo = _conv_out(width)
    wp = width_pad or width
    vp = wo_pad or wo
    cout, cin = w.shape[0], w.shape[1]
    scale = gamma / jnp.sqrt(var + _EPS)
    shift = beta + (b - mean) * scale
    wc = jnp.transpose(w, (2, 3, 1, 0)) * scale
    mask = np.zeros((5, wp, vp), np.float32)
    for j in range(5):
        mask[j, 2 * np.arange(wo) + j, np.arange(wo)] = 1.0
    spec = "jwv,ijco->icwvo" if c_major else "jwv,ijco->iwcvo"
    m = jnp.einsum(spec, jnp.asarray(mask), wc)
    m = m.reshape(5, (cin * wp) if c_major else (wp * cin),
                  vp * cout).astype(compute_dtype)
    shift_row = jnp.tile(shift, (vp,)).reshape(1, vp * cout).astype(jnp.float32)
    return m, shift_row


def _dqn_kernel_body(bt, c0, h0, hos, compute_dtype):
    ho1, ho2, ho3 = hos

    def body(x_ref, m1_ref, s1_ref, m2_ref, s2_ref,
             m3_ref, s3_ref, wh_ref, hb_ref, o_ref, scr1, scr2):
        xb = x_ref[0]
        w0 = xb.shape[3] // 2

        def chan_phase(p):
            parts = [xb[:, c, :, p * w0:(p + 1) * w0].astype(compute_dtype)
                     for c in range(c0)]
            return jnp.concatenate(parts, axis=2)

        def conv_layer(ae, ao, m_ref, s_ref, ho, out_dtype):
            acc = None
            for i in range(5):
                src = ae if i % 2 == 0 else ao
                sl = src[:, i // 2:i // 2 + ho, :]
                sl = sl.reshape(bt * ho, sl.shape[2])
                part = jnp.dot(sl, m_ref[i], preferred_element_type=jnp.float32)
                acc = part if acc is None else acc + part
            out = jnp.maximum(acc + s_ref[...], 0.0).astype(out_dtype)
            return out.reshape(bt, ho, out.shape[1])

        def phases(scr, hp, lanes):
            pe = scr[:, pl.Slice(0, hp, 2), :, :].astype(compute_dtype)
            po = scr[:, pl.Slice(1, hp, 2), :, :].astype(compute_dtype)
            return pe.reshape(bt, hp, lanes), po.reshape(bt, hp, lanes)

        a1 = conv_layer(chan_phase(0), chan_phase(1),
                        m1_ref, s1_ref, ho1, jnp.float32)
        n1 = a1.shape[2]
        scr1[...] = a1.reshape(bt, ho1, n1 // 128, 128)
        a2 = conv_layer(*phases(scr1, ho1 // 2, n1), m2_ref, s2_ref, ho2,
                        jnp.float32)
        n2 = a2.shape[2]
        scr2[...] = a2.reshape(bt, ho2, n2 // 128, 128)
        a3 = conv_layer(*phases(scr2, ho2 // 2, n2), m3_ref, s3_ref, ho3,
                        compute_dtype)

        q = None
        for r in range(ho3):
            part = jnp.dot(a3[:, r, :], wh_ref[r],
                           preferred_element_type=jnp.float32)
            q = part if q is None else q + part
        o_ref[...] = q + hb_ref[...]

    return body


def kernel(x,
           l1_w, l1_b, l1_gamma, l1_beta, l1_mean, l1_var,
           l2_w, l2_b, l2_gamma, l2_beta, l2_mean, l2_var,
           l3_w, l3_b, l3_gamma, l3_beta, l3_mean, l3_var,
           head_w, head_b, *, batch_tile=32, compute_dtype=jnp.bfloat16):
    N, C0, H0, W0 = x.shape
    BT = batch_tile if N % batch_tile == 0 else N
    NB = N // BT

    h1, w1 = _conv_out(H0), _conv_out(W0)
    h2, w2 = _conv_out(h1), _conv_out(w1)
    h3, w3 = _conv_out(h2), _conv_out(w2)
    c1, c2, c3 = l1_w.shape[0], l2_w.shape[0], l3_w.shape[0]
    n_act = head_w.shape[0]

    n1 = w1 * c1
    assert n1 % 128 == 0
    n2_req = w2 * c2
    w2p = w2
    while (w2p * c2) % 128 != 0:
        w2p += 1

    m1, s1 = _fold_layer(l1_w, l1_b, l1_gamma, l1_beta, l1_mean, l1_var,
                         W0, compute_dtype, c_major=True)
    m2, s2 = _fold_layer(l2_w, l2_b, l2_gamma, l2_beta, l2_mean, l2_var,
                         w1, compute_dtype, wo_pad=w2p)
    m3, s3 = _fold_layer(l3_w, l3_b, l3_gamma, l3_beta, l3_mean, l3_var,
                         w2, compute_dtype, width_pad=w2p)
    n2 = w2p * c2

    wh = (head_w.reshape(n_act, c3, h3, w3).transpose(2, 3, 1, 0)
          .reshape(h3, w3 * c3, n_act).astype(compute_dtype))
    hb = head_b.reshape(1, n_act).astype(jnp.float32)

    x5 = x.reshape(NB, BT, C0, H0 // 2, 2 * W0)

    body = _dqn_kernel_body(BT, C0, H0, (h1, h2, h3), compute_dtype)
    return pl.pallas_call(
        body,
        out_shape=jax.ShapeDtypeStruct((N, n_act), jnp.float32),
        grid=(NB,),
        in_specs=[
            pl.BlockSpec((1, BT, C0, H0 // 2, 2 * W0), lambda b: (b, 0, 0, 0, 0)),
            pl.BlockSpec(m1.shape, lambda b: (0, 0, 0)),
            pl.BlockSpec(s1.shape, lambda b: (0, 0)),
            pl.BlockSpec(m2.shape, lambda b: (0, 0, 0)),
            pl.BlockSpec(s2.shape, lambda b: (0, 0)),
            pl.BlockSpec(m3.shape, lambda b: (0, 0, 0)),
            pl.BlockSpec(s3.shape, lambda b: (0, 0)),
            pl.BlockSpec(wh.shape, lambda b: (0, 0, 0)),
            pl.BlockSpec(hb.shape, lambda b: (0, 0)),
        ],
        out_specs=pl.BlockSpec((BT, n_act), lambda b: (b, 0)),
        scratch_shapes=[
            pltpu.VMEM((BT, h1, n1 // 128, 128), jnp.float32),
            pltpu.VMEM((BT, h2, n2 // 128, 128), jnp.float32),
        ],
        compiler_params=pltpu.CompilerParams(
            dimension_semantics=("parallel",),
            vmem_limit_bytes=64 * 1024 * 1024),
    )(x5, m1, s1, m2, s2, m3, s3, wh, hb)


# speedup vs baseline: 1.1360x; 1.1360x over previous
import jax
import jax.numpy as jnp
import numpy as np
from jax.experimental import pallas as pl
from jax.experimental.pallas import tpu as pltpu

_EPS = 1e-5


def _conv_out(s):
    return (s - 5) // 2 + 1


def _fold_layer(w, b, gamma, beta, mean, var, width, compute_dtype,
                c_major=False, width_pad=None, wo_pad=None):
    wo = _conv_out(width)
    wp = width_pad or width
    vp = wo_pad or wo
    cout, cin = w.shape[0], w.shape[1]
    scale = gamma / jnp.sqrt(var + _EPS)
    shift = beta + (b - mean) * scale
    wc = jnp.transpose(w, (2, 3, 1, 0)) * scale
    mask = np.zeros((5, wp, vp), np.float32)
    for j in range(5):
        mask[j, 2 * np.arange(wo) + j, np.arange(wo)] = 1.0
    spec = "jwv,ijco->icwvo" if c_major else "jwv,ijco->iwcvo"
    m = jnp.einsum(spec, jnp.asarray(mask), wc)
    m = m.reshape(5, (cin * wp) if c_major else (wp * cin),
                  vp * cout).astype(compute_dtype)
    shift_row = jnp.tile(shift, (vp,)).reshape(1, vp * cout).astype(jnp.float32)
    return m, shift_row


def _dqn_kernel_body(bt, c0, h0, hos, compute_dtype):
    ho1, ho2, ho3 = hos

    def body(x_ref, m1_ref, s1_ref, m2_ref, s2_ref,
             m3_ref, s3_ref, wh_ref, hb_ref, o_ref, scr0, scr1, scr2):
        w0 = x_ref.shape[3]
        scr0[:, :, :, :w0] = x_ref[...]

        def chan_phase(p):
            parts = [scr0[:, c, pl.Slice(p, h0 // 2, 2), :][:, :, :w0]
                     .astype(compute_dtype) for c in range(c0)]
            return jnp.concatenate(parts, axis=2)

        def conv_layer(ae, ao, m_ref, s_ref, ho, out_dtype):
            acc = None
            for i in range(5):
                src = ae if i % 2 == 0 else ao
                sl = src[:, i // 2:i // 2 + ho, :]
                sl = sl.reshape(bt * ho, sl.shape[2])
                part = jnp.dot(sl, m_ref[i], preferred_element_type=jnp.float32)
                acc = part if acc is None else acc + part
            out = jnp.maximum(acc + s_ref[...], 0.0).astype(out_dtype)
            return out.reshape(bt, ho, out.shape[1])

        def phases(scr, hp, lanes):
            pe = scr[:, pl.Slice(0, hp, 2), :, :].astype(compute_dtype)
            po = scr[:, pl.Slice(1, hp, 2), :, :].astype(compute_dtype)
            return pe.reshape(bt, hp, lanes), po.reshape(bt, hp, lanes)

        a1 = conv_layer(chan_phase(0), chan_phase(1),
                        m1_ref, s1_ref, ho1, jnp.float32)
        n1 = a1.shape[2]
        scr1[...] = a1.reshape(bt, ho1, n1 // 128, 128)
        a2 = conv_layer(*phases(scr1, ho1 // 2, n1), m2_ref, s2_ref, ho2,
                        jnp.float32)
        n2 = a2.shape[2]
        scr2[...] = a2.reshape(bt, ho2, n2 // 128, 128)
        a3 = conv_layer(*phases(scr2, ho2 // 2, n2), m3_ref, s3_ref, ho3,
                        compute_dtype)

        q = None
        for r in range(ho3):
            part = jnp.dot(a3[:, r, :], wh_ref[r],
                           preferred_element_type=jnp.float32)
            q = part if q is None else q + part
        o_ref[...] = q + hb_ref[...]

    return body


def kernel(x,
           l1_w, l1_b, l1_gamma, l1_beta, l1_mean, l1_var,
           l2_w, l2_b, l2_gamma, l2_beta, l2_mean, l2_var,
           l3_w, l3_b, l3_gamma, l3_beta, l3_mean, l3_var,
           head_w, head_b, *, batch_tile=32, compute_dtype=jnp.bfloat16):
    N, C0, H0, W0 = x.shape
    BT = batch_tile if N % batch_tile == 0 else N
    NB = N // BT

    h1, w1 = _conv_out(H0), _conv_out(W0)
    h2, w2 = _conv_out(h1), _conv_out(w1)
    h3, w3 = _conv_out(h2), _conv_out(w2)
    c1, c2, c3 = l1_w.shape[0], l2_w.shape[0], l3_w.shape[0]
    n_act = head_w.shape[0]

    n1 = w1 * c1
    assert n1 % 128 == 0
    n2_req = w2 * c2
    w2p = w2
    while (w2p * c2) % 128 != 0:
        w2p += 1

    m1, s1 = _fold_layer(l1_w, l1_b, l1_gamma, l1_beta, l1_mean, l1_var,
                         W0, compute_dtype, c_major=True)
    m2, s2 = _fold_layer(l2_w, l2_b, l2_gamma, l2_beta, l2_mean, l2_var,
                         w1, compute_dtype, wo_pad=w2p)
    m3, s3 = _fold_layer(l3_w, l3_b, l3_gamma, l3_beta, l3_mean, l3_var,
                         w2, compute_dtype, width_pad=w2p)
    n2 = w2p * c2

    wh = (head_w.reshape(n_act, c3, h3, w3).transpose(2, 3, 1, 0)
          .reshape(h3, w3 * c3, n_act).astype(compute_dtype))
    hb = head_b.reshape(1, n_act).astype(jnp.float32)

    body = _dqn_kernel_body(BT, C0, H0, (h1, h2, h3), compute_dtype)
    return pl.pallas_call(
        body,
        out_shape=jax.ShapeDtypeStruct((N, n_act), jnp.float32),
        grid=(NB,),
        in_specs=[
            pl.BlockSpec((BT, C0, H0, W0), lambda b: (b, 0, 0, 0)),
            pl.BlockSpec(m1.shape, lambda b: (0, 0, 0)),
            pl.BlockSpec(s1.shape, lambda b: (0, 0)),
            pl.BlockSpec(m2.shape, lambda b: (0, 0, 0)),
            pl.BlockSpec(s2.shape, lambda b: (0, 0)),
            pl.BlockSpec(m3.shape, lambda b: (0, 0, 0)),
            pl.BlockSpec(s3.shape, lambda b: (0, 0)),
            pl.BlockSpec(wh.shape, lambda b: (0, 0, 0)),
            pl.BlockSpec(hb.shape, lambda b: (0, 0)),
        ],
        out_specs=pl.BlockSpec((BT, n_act), lambda b: (b, 0)),
        scratch_shapes=[
            pltpu.VMEM((BT, C0, H0, 128), jnp.float32),
            pltpu.VMEM((BT, h1, n1 // 128, 128), jnp.float32),
            pltpu.VMEM((BT, h2, n2 // 128, 128), jnp.float32),
        ],
        compiler_params=pltpu.CompilerParams(
            dimension_semantics=("parallel",),
            vmem_limit_bytes=64 * 1024 * 1024),
    )(x, m1, s1, m2, s2, m3, s3, wh, hb)


# speedup vs baseline: 1.1418x; 1.0051x over previous
import jax
import jax.numpy as jnp
import numpy as np
from jax.experimental import pallas as pl
from jax.experimental.pallas import tpu as pltpu

_EPS = 1e-5


def _conv_out(s):
    return (s - 5) // 2 + 1


def _fold_layer(w, b, gamma, beta, mean, var, width, compute_dtype,
                c_major=False, width_pad=None, wo_pad=None):
    wo = _conv_out(width)
    wp = width_pad or width
    vp = wo_pad or wo
    cout, cin = w.shape[0], w.shape[1]
    scale = gamma / jnp.sqrt(var + _EPS)
    shift = beta + (b - mean) * scale
    wc = jnp.transpose(w, (2, 3, 1, 0)) * scale
    mask = np.zeros((5, wp, vp), np.float32)
    for j in range(5):
        mask[j, 2 * np.arange(wo) + j, np.arange(wo)] = 1.0
    spec = "jwv,ijco->icwvo" if c_major else "jwv,ijco->iwcvo"
    m = jnp.einsum(spec, jnp.asarray(mask), wc)
    m = m.reshape(5, (cin * wp) if c_major else (wp * cin),
                  vp * cout).astype(compute_dtype)
    shift_row = jnp.tile(shift, (vp,)).reshape(1, vp * cout).astype(jnp.float32)
    return m, shift_row


def _dqn_kernel_body(bt, c0, h0, hos, compute_dtype):
    ho1, ho2, ho3 = hos

    def body(x_ref, m1_ref, s1_ref, m2_ref, s2_ref,
             m3_ref, s3_ref, wh_ref, hb_ref, o_ref, scr0, scr1, scr2):
        w0 = x_ref.shape[3]
        scr0[:, :, :, :w0] = x_ref[...]

        def chan_phase(p):
            parts = [scr0[:, c, pl.Slice(p, h0 // 2, 2), :][:, :, :w0]
                     .astype(compute_dtype) for c in range(c0)]
            return jnp.concatenate(parts, axis=2)

        def conv_layer(ae, ao, m_ref, s_ref, ho, out_dtype):
            acc = None
            for i in range(5):
                src = ae if i % 2 == 0 else ao
                sl = src[:, i // 2:i // 2 + ho, :]
                sl = sl.reshape(bt * ho, sl.shape[2])
                part = jnp.dot(sl, m_ref[i], preferred_element_type=jnp.float32)
                acc = part if acc is None else acc + part
            out = jnp.maximum(acc + s_ref[...], 0.0).astype(out_dtype)
            return out.reshape(bt, ho, out.shape[1])

        def phases(scr, hp, lanes):
            pe = scr[:, pl.Slice(0, hp, 2), :, :].astype(compute_dtype)
            po = scr[:, pl.Slice(1, hp, 2), :, :].astype(compute_dtype)
            return pe.reshape(bt, hp, lanes), po.reshape(bt, hp, lanes)

        a1 = conv_layer(chan_phase(0), chan_phase(1),
                        m1_ref, s1_ref, ho1, jnp.float32)
        n1 = a1.shape[2]
        scr1[...] = a1.reshape(bt, ho1, n1 // 128, 128)
        a2 = conv_layer(*phases(scr1, ho1 // 2, n1), m2_ref, s2_ref, ho2,
                        jnp.float32)
        n2 = a2.shape[2]
        scr2[...] = a2.reshape(bt, ho2, n2 // 128, 128)
        a3 = conv_layer(*phases(scr2, ho2 // 2, n2), m3_ref, s3_ref, ho3,
                        compute_dtype)

        q = None
        for r in range(ho3):
            part = jnp.dot(a3[:, r, :], wh_ref[r],
                           preferred_element_type=jnp.float32)
            q = part if q is None else q + part
        o_ref[...] = q + hb_ref[...]

    return body


def kernel(x,
           l1_w, l1_b, l1_gamma, l1_beta, l1_mean, l1_var,
           l2_w, l2_b, l2_gamma, l2_beta, l2_mean, l2_var,
           l3_w, l3_b, l3_gamma, l3_beta, l3_mean, l3_var,
           head_w, head_b, *, batch_tile=64, compute_dtype=jnp.bfloat16):
    N, C0, H0, W0 = x.shape
    BT = batch_tile if N % batch_tile == 0 else N
    NB = N // BT

    h1, w1 = _conv_out(H0), _conv_out(W0)
    h2, w2 = _conv_out(h1), _conv_out(w1)
    h3, w3 = _conv_out(h2), _conv_out(w2)
    c1, c2, c3 = l1_w.shape[0], l2_w.shape[0], l3_w.shape[0]
    n_act = head_w.shape[0]

    n1 = w1 * c1
    assert n1 % 128 == 0
    n2_req = w2 * c2
    w2p = w2
    while (w2p * c2) % 128 != 0:
        w2p += 1

    m1, s1 = _fold_layer(l1_w, l1_b, l1_gamma, l1_beta, l1_mean, l1_var,
                         W0, compute_dtype, c_major=True)
    m2, s2 = _fold_layer(l2_w, l2_b, l2_gamma, l2_beta, l2_mean, l2_var,
                         w1, compute_dtype, wo_pad=w2p)
    m3, s3 = _fold_layer(l3_w, l3_b, l3_gamma, l3_beta, l3_mean, l3_var,
                         w2, compute_dtype, width_pad=w2p)
    n2 = w2p * c2

    wh = (head_w.reshape(n_act, c3, h3, w3).transpose(2, 3, 1, 0)
          .reshape(h3, w3 * c3, n_act).astype(compute_dtype))
    hb = head_b.reshape(1, n_act).astype(jnp.float32)

    body = _dqn_kernel_body(BT, C0, H0, (h1, h2, h3), compute_dtype)
    return pl.pallas_call(
        body,
        out_shape=jax.ShapeDtypeStruct((N, n_act), jnp.float32),
        grid=(NB,),
        in_specs=[
            pl.BlockSpec((BT, C0, H0, W0), lambda b: (b, 0, 0, 0)),
            pl.BlockSpec(m1.shape, lambda b: (0, 0, 0)),
            pl.BlockSpec(s1.shape, lambda b: (0, 0)),
            pl.BlockSpec(m2.shape, lambda b: (0, 0, 0)),
            pl.BlockSpec(s2.shape, lambda b: (0, 0)),
            pl.BlockSpec(m3.shape, lambda b: (0, 0, 0)),
            pl.BlockSpec(s3.shape, lambda b: (0, 0)),
            pl.BlockSpec(wh.shape, lambda b: (0, 0, 0)),
            pl.BlockSpec(hb.shape, lambda b: (0, 0)),
        ],
        out_specs=pl.BlockSpec((BT, n_act), lambda b: (b, 0)),
        scratch_shapes=[
            pltpu.VMEM((BT, C0, H0, 128), jnp.float32),
            pltpu.VMEM((BT, h1, n1 // 128, 128), jnp.float32),
            pltpu.VMEM((BT, h2, n2 // 128, 128), jnp.float32),
        ],
        compiler_params=pltpu.CompilerParams(
            dimension_semantics=("parallel",),
            vmem_limit_bytes=64 * 1024 * 1024),
    )(x, m1, s1, m2, s2, m3, s3, wh, hb)


# speedup vs baseline: 1.2047x; 1.0551x over previous
import jax
import jax.numpy as jnp
import numpy as np
from jax.experimental import pallas as pl
from jax.experimental.pallas import tpu as pltpu

_EPS = 1e-5


def _conv_out(s):
    return (s - 5) // 2 + 1


def _fold_layer(w, b, gamma, beta, mean, var, width, compute_dtype,
                c_major=False, width_pad=None, wo_pad=None):
    wo = _conv_out(width)
    wp = width_pad or width
    vp = wo_pad or wo
    cout, cin = w.shape[0], w.shape[1]
    scale = gamma / jnp.sqrt(var + _EPS)
    shift = beta + (b - mean) * scale
    wc = jnp.transpose(w, (2, 3, 1, 0)) * scale
    mask = np.zeros((5, wp, vp), np.float32)
    for j in range(5):
        mask[j, 2 * np.arange(wo) + j, np.arange(wo)] = 1.0
    spec = "jwv,ijco->icwvo" if c_major else "jwv,ijco->iwcvo"
    m = jnp.einsum(spec, jnp.asarray(mask), wc)
    m = m.reshape(5, (cin * wp) if c_major else (wp * cin),
                  vp * cout).astype(compute_dtype)
    shift_row = jnp.tile(shift, (vp,)).reshape(1, vp * cout).astype(jnp.float32)
    return m, shift_row


def _dqn_kernel_body(bt, c0, h0, hos, compute_dtype):
    ho1, ho2, ho3 = hos

    def body(x_ref, m1_ref, s1_ref, m2_ref, s2_ref,
             m3_ref, s3_ref, wh_ref, hb_ref, o_ref, scr0):
        w0 = x_ref.shape[3]
        scr0[:, :, :, :w0] = x_ref[...]

        nt1 = ho1 // 4
        PH = []
        for s in range(2 * 3 + 5):
            parts = [scr0[:, c, pl.Slice(s, nt1, 8), :][:, :, :w0]
                     .astype(compute_dtype) for c in range(c0)]
            PH.append(jnp.concatenate(parts, axis=2))

        def dot_taps(slices, m_ref, s_ref, mrows, lanes, out_dtype):
            acc = None
            for i, sl in enumerate(slices):
                sl = sl.reshape(mrows, lanes)
                part = jnp.dot(sl, m_ref[i], preferred_element_type=jnp.float32)
                acc = part if acc is None else acc + part
            out = jnp.maximum(acc + s_ref[...], 0.0).astype(out_dtype)
            return out.reshape(bt, mrows // bt, out.shape[1])

        g1 = [dot_taps([PH[2 * g + i] for i in range(5)],
                       m1_ref, s1_ref, bt * nt1, w0 * c0, compute_dtype)
              for g in range(4)]
        n1 = g1[0].shape[2]

        nt2 = ho2 // 2
        g2 = [dot_taps([g1[(2 * p + i) % 4][:, (2 * p + i) // 4:
                                            (2 * p + i) // 4 + nt2, :]
                        for i in range(5)],
                       m2_ref, s2_ref, bt * nt2, n1, compute_dtype)
              for p in range(2)]
        n2 = g2[0].shape[2]

        a3 = dot_taps([g2[i % 2][:, i // 2:i // 2 + ho3, :] for i in range(5)],
                      m3_ref, s3_ref, bt * ho3, n2, compute_dtype)

        q = None
        for r in range(ho3):
            part = jnp.dot(a3[:, r, :], wh_ref[r],
                           preferred_element_type=jnp.float32)
            q = part if q is None else q + part
        o_ref[...] = q + hb_ref[...]

    return body


def kernel(x,
           l1_w, l1_b, l1_gamma, l1_beta, l1_mean, l1_var,
           l2_w, l2_b, l2_gamma, l2_beta, l2_mean, l2_var,
           l3_w, l3_b, l3_gamma, l3_beta, l3_mean, l3_var,
           head_w, head_b, *, batch_tile=64, compute_dtype=jnp.bfloat16):
    N, C0, H0, W0 = x.shape
    BT = batch_tile if N % batch_tile == 0 else N
    NB = N // BT

    h1, w1 = _conv_out(H0), _conv_out(W0)
    h2, w2 = _conv_out(h1), _conv_out(w1)
    h3, w3 = _conv_out(h2), _conv_out(w2)
    c1, c2, c3 = l1_w.shape[0], l2_w.shape[0], l3_w.shape[0]
    n_act = head_w.shape[0]

    m1, s1 = _fold_layer(l1_w, l1_b, l1_gamma, l1_beta, l1_mean, l1_var,
                         W0, compute_dtype, c_major=True)
    m2, s2 = _fold_layer(l2_w, l2_b, l2_gamma, l2_beta, l2_mean, l2_var,
                         w1, compute_dtype)
    m3, s3 = _fold_layer(l3_w, l3_b, l3_gamma, l3_beta, l3_mean, l3_var,
                         w2, compute_dtype)

    wh = (head_w.reshape(n_act, c3, h3, w3).transpose(2, 3, 1, 0)
          .reshape(h3, w3 * c3, n_act).astype(compute_dtype))
    hb = head_b.reshape(1, n_act).astype(jnp.float32)

    body = _dqn_kernel_body(BT, C0, H0, (h1, h2, h3), compute_dtype)
    return pl.pallas_call(
        body,
        out_shape=jax.ShapeDtypeStruct((N, n_act), jnp.float32),
        grid=(NB,),
        in_specs=[
            pl.BlockSpec((BT, C0, H0, W0), lambda b: (b, 0, 0, 0)),
            pl.BlockSpec(m1.shape, lambda b: (0, 0, 0)),
            pl.BlockSpec(s1.shape, lambda b: (0, 0)),
            pl.BlockSpec(m2.shape, lambda b: (0, 0, 0)),
            pl.BlockSpec(s2.shape, lambda b: (0, 0)),
            pl.BlockSpec(m3.shape, lambda b: (0, 0, 0)),
            pl.BlockSpec(s3.shape, lambda b: (0, 0)),
            pl.BlockSpec(wh.shape, lambda b: (0, 0, 0)),
            pl.BlockSpec(hb.shape, lambda b: (0, 0)),
        ],
        out_specs=pl.BlockSpec((BT, n_act), lambda b: (b, 0)),
        scratch_shapes=[
            pltpu.VMEM((BT, C0, H0, 128), jnp.float32),
        ],
        compiler_params=pltpu.CompilerParams(
            dimension_semantics=("parallel",),
            vmem_limit_bytes=64 * 1024 * 1024),
    )(x, m1, s1, m2, s2, m3, s3, wh, hb)
